# baseline (speedup 1.0000x reference)
import jax
import jax.numpy as jnp
from jax import lax
from jax.experimental import pallas as pl
from jax.experimental.pallas import tpu as pltpu
from jax.experimental.pallas import tpu_sc as plsc

DIM = 64
BATCH = 4096
N_USERS = 1872
RATERS = 32
NUM_CORES = 2
NUM_SUBCORES = 16
NUM_WORKERS = NUM_CORES * NUM_SUBCORES
BPW = BATCH // NUM_WORKERS
LANES = 16
NBUF = 3
STRIPE = N_USERS // NUM_SUBCORES


def _body(v_hbm, iu_hbm, tab_hbm, out_hbm, v_v, raters_v, buf, out_st,
          stripe_v, packed_v, tab_sh, sem_in, sem_g):
    sid = lax.axis_index("s")
    wid = sid * NUM_CORES + lax.axis_index("c")
    base = wid * BPW

    pltpu.sync_copy(v_hbm.at[pl.ds(base, BPW)], v_v)
    iu_copy = pltpu.async_copy(iu_hbm.at[v_v], raters_v, sem_in)

    row0 = sid * STRIPE
    pltpu.sync_copy(tab_hbm.at[pl.ds(row0, STRIPE)], stripe_v)
    rnd = jnp.full((LANES,), 0x8000, dtype=jnp.int32)

    def pack_row(i, carry):
        w = [plsc.bitcast(stripe_v[i, pl.ds(k * LANES, LANES)], jnp.int32)
             for k in range(4)]
        for h in range(2):
            lo = lax.shift_right_logical(w[2 * h] + rnd, 16)
            hi = (w[2 * h + 1] + rnd) & jnp.full((LANES,), -65536, jnp.int32)
            packed_v[i, pl.ds(h * LANES, LANES)] = lo | hi
        return carry

    lax.fori_loop(0, STRIPE, pack_row, 0, unroll=4)
    pltpu.sync_copy(packed_v, tab_sh.at[pl.ds(row0, STRIPE)])

    plsc.subcore_barrier()
    iu_copy.wait()

    def fire(c):
        par = lax.rem(c, NBUF)
        pltpu.async_copy(
            tab_sh.at[raters_v.at[c]],
            buf.at[pl.ds(par * RATERS, RATERS)],
            sem_g)

    for p in range(NBUF):
        fire(jnp.int32(p))

    himask = jnp.full((LANES,), -65536, dtype=jnp.int32)

    def chunk_body(c, carry):
        par = lax.rem(c, NBUF)
        rbase = par * RATERS
        pltpu.make_async_copy(
            iu_hbm.at[pl.ds(0, RATERS)],
            buf.at[pl.ds(rbase, RATERS)],
            sem_g).wait()
        acc = [jnp.zeros((LANES,), jnp.float32) for _ in range(4)]
        for r in range(RATERS):
            row = rbase + r
            w0 = buf[row, pl.ds(0, LANES)]
            w1 = buf[row, pl.ds(LANES, LANES)]
            acc[0] = acc[0] + plsc.bitcast(w0 << 16, jnp.float32)
            acc[1] = acc[1] + plsc.bitcast(w0 & himask, jnp.float32)
            acc[2] = acc[2] + plsc.bitcast(w1 << 16, jnp.float32)
            acc[3] = acc[3] + plsc.bitcast(w1 & himask, jnp.float32)
        for k in range(4):
            out_st[c, pl.ds(k * LANES, LANES)] = acc[k]

        @pl.when(c + NBUF < BPW)
        def _():
            fire(c + NBUF)

        return carry

    lax.fori_loop(0, BPW, chunk_body, 0, unroll=2)

    pltpu.sync_copy(out_st, out_hbm.at[pl.ds(base, BPW)])


@jax.jit
def _ucn_sc(v, item_users, U_table):
    mesh = plsc.VectorSubcoreMesh(core_axis_name="c", subcore_axis_name="s")
    return pl.kernel(
        _body,
        out_type=jax.ShapeDtypeStruct((BATCH, DIM), jnp.float32),
        mesh=mesh,
        compiler_params=pltpu.CompilerParams(
            needs_layout_passes=False, use_tc_tiling_on_sc=False),
        scratch_types=[
            pltpu.VMEM((BPW,), jnp.int32),
            pltpu.VMEM((BPW, RATERS), jnp.int32),
            pltpu.VMEM((NBUF * RATERS, DIM // 2), jnp.int32),
            pltpu.VMEM((BPW, DIM), jnp.float32),
            pltpu.VMEM((STRIPE, DIM), jnp.float32),
            pltpu.VMEM((STRIPE, DIM // 2), jnp.int32),
            pltpu.VMEM_SHARED((N_USERS, DIM // 2), jnp.int32),
            pltpu.SemaphoreType.DMA,
            pltpu.SemaphoreType.DMA,
        ],
    )(v, item_users, U_table)


def kernel(u, v, item_users, U_table):
    del u
    return _ucn_sc(v, item_users, U_table)


# speedup vs baseline: 1.0077x; 1.0077x over previous
import numpy as np

import jax
import jax.numpy as jnp
from jax import lax
from jax.experimental import pallas as pl
from jax.experimental.pallas import tpu as pltpu
from jax.experimental.pallas import tpu_sc as plsc

DIM = 64
BATCH = 4096
N_USERS = 1872
RATERS = 32
NUM_CORES = 2
NUM_SUBCORES = 16
NUM_WORKERS = NUM_CORES * NUM_SUBCORES
BPW = BATCH // NUM_WORKERS
LANES = 16
NBUF = 3

_CP = np.zeros(DIM, np.int32)
for _k in range(LANES):
    _CP[2 * _k] = _k
    _CP[2 * _k + 1] = LANES + _k
    _CP[2 * LANES + 2 * _k] = 2 * LANES + _k
    _CP[2 * LANES + 2 * _k + 1] = 3 * LANES + _k
_COLPERM = tuple(int(x) for x in _CP)


def _body(v_hbm, iu_hbm, tabw_hbm, out_hbm, v_v, raters_v, buf, out_st,
          tab_sh, sem_in, sem_g):
    sid = lax.axis_index("s")
    wid = sid * NUM_CORES + lax.axis_index("c")
    base = wid * BPW

    pltpu.sync_copy(v_hbm.at[pl.ds(base, BPW)], v_v)
    iu_copy = pltpu.async_copy(iu_hbm.at[v_v], raters_v, sem_in)

    @pl.when(sid == 0)
    def _():
        pltpu.sync_copy(tabw_hbm, tab_sh)

    plsc.subcore_barrier()
    iu_copy.wait()

    def fire(c):
        par = lax.rem(c, NBUF)
        pltpu.async_copy(
            tab_sh.at[raters_v.at[c]],
            buf.at[pl.ds(par * RATERS, RATERS)],
            sem_g)

    for p in range(NBUF):
        fire(jnp.int32(p))

    himask = jnp.full((LANES,), -65536, dtype=jnp.int32)

    def chunk_body(c, carry):
        par = lax.rem(c, NBUF)
        rbase = par * RATERS
        pltpu.make_async_copy(
            tabw_hbm.at[pl.ds(0, RATERS)],
            buf.at[pl.ds(rbase, RATERS)],
            sem_g).wait()
        acc = [jnp.zeros((LANES,), jnp.float32) for _ in range(4)]
        for r in range(RATERS):
            row = rbase + r
            w0 = buf[row, pl.ds(0, LANES)]
            w1 = buf[row, pl.ds(LANES, LANES)]
            acc[0] = acc[0] + plsc.bitcast(w0 << 16, jnp.float32)
            acc[1] = acc[1] + plsc.bitcast(w0 & himask, jnp.float32)
            acc[2] = acc[2] + plsc.bitcast(w1 << 16, jnp.float32)
            acc[3] = acc[3] + plsc.bitcast(w1 & himask, jnp.float32)
        for k in range(4):
            out_st[c, pl.ds(k * LANES, LANES)] = acc[k]

        @pl.when(c + NBUF < BPW)
        def _():
            fire(c + NBUF)

        return carry

    lax.fori_loop(0, BPW, chunk_body, 0, unroll=2)

    pltpu.sync_copy(out_st, out_hbm.at[pl.ds(base, BPW)])


@jax.jit
def _ucn_sc(v, item_users, tab_w):
    mesh = plsc.VectorSubcoreMesh(core_axis_name="c", subcore_axis_name="s")
    return pl.kernel(
        _body,
        out_type=jax.ShapeDtypeStruct((BATCH, DIM), jnp.float32),
        mesh=mesh,
        compiler_params=pltpu.CompilerParams(
            needs_layout_passes=False, use_tc_tiling_on_sc=False),
        scratch_types=[
            pltpu.VMEM((BPW,), jnp.int32),
            pltpu.VMEM((BPW, RATERS), jnp.int32),
            pltpu.VMEM((NBUF * RATERS, DIM // 2), jnp.int32),
            pltpu.VMEM((BPW, DIM), jnp.float32),
            pltpu.VMEM_SHARED((N_USERS, DIM // 2), jnp.int32),
            pltpu.SemaphoreType.DMA,
            pltpu.SemaphoreType.DMA,
        ],
    )(v, item_users, tab_w)


def kernel(u, v, item_users, U_table):
    del u
    u_bf = U_table[:, _COLPERM].astype(jnp.bfloat16)
    tab_w = jax.lax.bitcast_convert_type(
        u_bf.reshape(N_USERS, DIM // 2, 2), jnp.int32)
    return _ucn_sc(v, item_users, tab_w)


# speedup vs baseline: 1.0207x; 1.0129x over previous
import numpy as np

import jax
import jax.numpy as jnp
from jax import lax
from jax.experimental import pallas as pl
from jax.experimental.pallas import tpu as pltpu
from jax.experimental.pallas import tpu_sc as plsc

DIM = 64
BATCH = 4096
N_USERS = 1872
RATERS = 32
NUM_CORES = 2
NUM_SUBCORES = 16
NUM_WORKERS = NUM_CORES * NUM_SUBCORES
BPW = BATCH // NUM_WORKERS
LANES = 16
NBUF = 4

_CP = np.zeros(DIM, np.int32)
for _k in range(LANES):
    _CP[2 * _k] = _k
    _CP[2 * _k + 1] = LANES + _k
    _CP[2 * LANES + 2 * _k] = 2 * LANES + _k
    _CP[2 * LANES + 2 * _k + 1] = 3 * LANES + _k
_COLPERM = tuple(int(x) for x in _CP)


def _body(v_hbm, iu_hbm, tabw_hbm, out_hbm, v_v, raters_v, buf, out_st,
          tab_sh, sem_in, sem_g):
    sid = lax.axis_index("s")
    wid = sid * NUM_CORES + lax.axis_index("c")
    base = wid * BPW

    pltpu.sync_copy(v_hbm.at[pl.ds(base, BPW)], v_v)
    iu_copy = pltpu.async_copy(iu_hbm.at[v_v], raters_v, sem_in)

    @pl.when(sid == 0)
    def _():
        pltpu.sync_copy(tabw_hbm, tab_sh)

    plsc.subcore_barrier()
    iu_copy.wait()

    def fire(c):
        par = lax.rem(c, NBUF)
        pltpu.async_copy(
            tab_sh.at[raters_v.at[c]],
            buf.at[pl.ds(par * RATERS, RATERS)],
            sem_g)

    for p in range(NBUF):
        fire(jnp.int32(p))

    himask = jnp.full((LANES,), -65536, dtype=jnp.int32)

    def chunk_body(c, carry):
        par = lax.rem(c, NBUF)
        rbase = par * RATERS
        pltpu.make_async_copy(
            tabw_hbm.at[pl.ds(0, RATERS)],
            buf.at[pl.ds(rbase, RATERS)],
            sem_g).wait()
        acc = [jnp.zeros((LANES,), jnp.float32) for _ in range(4)]
        for r in range(RATERS):
            row = rbase + r
            w0 = buf[row, pl.ds(0, LANES)]
            w1 = buf[row, pl.ds(LANES, LANES)]
            acc[0] = acc[0] + plsc.bitcast(w0 << 16, jnp.float32)
            acc[1] = acc[1] + plsc.bitcast(w0 & himask, jnp.float32)
            acc[2] = acc[2] + plsc.bitcast(w1 << 16, jnp.float32)
            acc[3] = acc[3] + plsc.bitcast(w1 & himask, jnp.float32)
        for k in range(4):
            out_st[c, pl.ds(k * LANES, LANES)] = acc[k]

        @pl.when(c + NBUF < BPW)
        def _():
            fire(c + NBUF)

        return carry

    lax.fori_loop(0, BPW, chunk_body, 0, unroll=2)

    pltpu.sync_copy(out_st, out_hbm.at[pl.ds(base, BPW)])


@jax.jit
def _ucn_sc(v, item_users, tab_w):
    mesh = plsc.VectorSubcoreMesh(core_axis_name="c", subcore_axis_name="s")
    return pl.kernel(
        _body,
        out_type=jax.ShapeDtypeStruct((BATCH, DIM), jnp.float32),
        mesh=mesh,
        compiler_params=pltpu.CompilerParams(
            needs_layout_passes=False, use_tc_tiling_on_sc=False),
        scratch_types=[
            pltpu.VMEM((BPW,), jnp.int32),
            pltpu.VMEM((BPW, RATERS), jnp.int32),
            pltpu.VMEM((NBUF * RATERS, DIM // 2), jnp.int32),
            pltpu.VMEM((BPW, DIM), jnp.float32),
            pltpu.VMEM_SHARED((N_USERS, DIM // 2), jnp.int32),
            pltpu.SemaphoreType.DMA,
            pltpu.SemaphoreType.DMA,
        ],
    )(v, item_users, tab_w)


def kernel(u, v, item_users, U_table):
    del u
    u_bf = U_table[:, _COLPERM].astype(jnp.bfloat16)
    tab_w = jax.lax.bitcast_convert_type(
        u_bf.reshape(N_USERS, DIM // 2, 2), jnp.int32)
    return _ucn_sc(v, item_users, tab_w)
